# Initial kernel scaffold; baseline (speedup 1.0000x reference)
import functools

import jax
import jax.numpy as jnp
from jax import lax
from jax.experimental import pallas as pl
from jax.experimental.pallas import tpu as pltpu
from jax.experimental.pallas import tpu_sc as plsc

_F32 = jnp.float32
_I32 = jnp.int32


def _tc_a(x, W1, a1, a2):
    n, f_in = x.shape
    hid = W1.shape[1]
    blk = 5000
    grid = n // blk

    def body(x_ref, w_ref, s_ref, d_ref, h_ref, as_ref, ad_ref):
        h = lax.dot_general(x_ref[...], w_ref[...], (((1,), (0,)), ((), ())),
                            preferred_element_type=_F32)
        h_ref[...] = h
        as_ref[...] = jnp.sum(h * s_ref[...][None, :], axis=1)
        ad_ref[...] = jnp.sum(h * d_ref[...][None, :], axis=1)

    return pl.pallas_call(
        body,
        grid=(grid,),
        in_specs=[pl.BlockSpec((blk, f_in), lambda i: (i, 0)),
                  pl.BlockSpec((f_in, hid), lambda i: (0, 0)),
                  pl.BlockSpec((hid,), lambda i: (0,)),
                  pl.BlockSpec((hid,), lambda i: (0,))],
        out_specs=[pl.BlockSpec((blk, hid), lambda i: (i, 0)),
                   pl.BlockSpec((blk,), lambda i: (i,)),
                   pl.BlockSpec((blk,), lambda i: (i,))],
        out_shape=[jax.ShapeDtypeStruct((n, hid), _F32),
                   jax.ShapeDtypeStruct((n,), _F32),
                   jax.ShapeDtypeStruct((n,), _F32)],
    )(x, W1, a1, a2)


def _tc_b(num, den, b1, W2):
    n = den.shape[0]
    hid = b1.shape[0]
    blk = 5000
    grid = n // blk

    def body(num_ref, den_ref, b_ref, w_ref, h2_ref):
        hcat = jnp.concatenate([num_ref[0], num_ref[1]], axis=1)
        d = den_ref[...][:, None] + 1e-16
        hr = jnp.maximum(hcat / d + b_ref[...][None, :], 0.0)
        h2_ref[...] = jnp.sum(hr * w_ref[...][:, 0][None, :], axis=1)

    return pl.pallas_call(
        body,
        grid=(grid,),
        in_specs=[pl.BlockSpec((2, blk, 16), lambda i: (0, i, 0)),
                  pl.BlockSpec((blk,), lambda i: (i,)),
                  pl.BlockSpec((hid,), lambda i: (0,)),
                  pl.BlockSpec((hid, 1), lambda i: (0, 0))],
        out_specs=[pl.BlockSpec((blk,), lambda i: (i,))],
        out_shape=[jax.ShapeDtypeStruct((n,), _F32)],
    )(num, den, b1, W2)[0]


def _tc_c(num2, den2, b2):
    n = num2.shape[1]
    blk = 5000
    grid = n // blk

    def body(n_ref, d_ref, b_ref, o_ref):
        v = (n_ref[0] + n_ref[1]) / (d_ref[0] + d_ref[1] + 1e-16)
        o_ref[...] = v[:, None] + b_ref[...][None, :]

    return pl.pallas_call(
        body,
        grid=(grid,),
        in_specs=[pl.BlockSpec((2, blk), lambda i: (0, i)),
                  pl.BlockSpec((2, blk), lambda i: (0, i)),
                  pl.BlockSpec((1,), lambda i: (0,))],
        out_specs=[pl.BlockSpec((blk, 1), lambda i: (i, 0))],
        out_shape=[jax.ShapeDtypeStruct((n, 1), _F32)],
    )(num2, den2, b2)[0]


_LANES = 16


def _sc1(src2d, dst2d, as1, ad1, hrows, n, n_pad, tw):
    rows_tot = src2d.shape[0]
    rpt = rows_tot // 16
    ch = 17
    n_outer = rpt // ch
    assert n_outer * ch == rpt
    zr = rpt
    mesh = plsc.VectorSubcoreMesh(core_axis_name="c", subcore_axis_name="s")

    @functools.partial(
        pl.kernel, mesh=mesh,
        out_type=[jax.ShapeDtypeStruct((2 * n, 16), _F32),
                  jax.ShapeDtypeStruct((n,), _F32)],
        scratch_types=[
            pltpu.VMEM_SHARED((n_pad, 16), _F32),
            pltpu.VMEM_SHARED((n_pad,), _F32),
            pltpu.VMEM_SHARED((n_pad,), _F32),
            pltpu.VMEM_SHARED((n_pad,), _F32),
            pltpu.VMEM((ch, 128), _I32),
            pltpu.VMEM((ch, 128), _I32),
            pltpu.VMEM((ch, 128), _I32),
            pltpu.VMEM((ch * 128,), _F32),
            pltpu.VMEM((ch * 128,), _F32),
            pltpu.VMEM((ch * 128,), _F32),
            pltpu.VMEM((ch * 128, 16), _F32),
            pltpu.VMEM((zr, 16), _F32),
            pltpu.VMEM((tw,), _F32),
            pltpu.SemaphoreType.DMA,
        ])
    def k(src_hbm, dst_hbm, as_hbm, ad_hbm, hrows_hbm, num_out, den_out,
          acc, den, as_sp, ad_sp, isrc, idst, irow, asg, adg, wbuf, rows,
          z2, z1, sem):
        c = lax.axis_index("c")
        s = lax.axis_index("s")
        zv = jnp.zeros((_LANES,), _F32)

        def z2_body(i, _):
            z2[i, :] = zv
            return 0
        lax.fori_loop(0, zr, z2_body, 0)

        def z1_body(i, _):
            z1[pl.ds(i * _LANES, _LANES)] = zv
            return 0
        lax.fori_loop(0, tw // _LANES, z1_body, 0)

        base = s * tw
        for b in range(8):
            pltpu.sync_copy(z2, acc.at[pl.ds(base + b * zr, zr)])
        pltpu.sync_copy(z1, den.at[pl.ds(base, tw)])

        off = jnp.minimum(s * tw, n - tw)
        pltpu.sync_copy(as_hbm.at[pl.ds(off, tw)], as_sp.at[pl.ds(off, tw)])
        pltpu.sync_copy(ad_hbm.at[pl.ds(off, tw)], ad_sp.at[pl.ds(off, tw)])

        @pl.when(s == 0)
        def _():
            pad = n_pad - n
            pltpu.sync_copy(z1.at[pl.ds(0, pad)], as_sp.at[pl.ds(n, pad)])
            pltpu.sync_copy(z1.at[pl.ds(0, pad)], ad_sp.at[pl.ds(n, pad)])

        plsc.subcore_barrier()

        def outer(o, _):
            r0 = s * rpt + o * ch
            pltpu.sync_copy(src_hbm.at[pl.ds(r0, ch)], isrc)
            pltpu.sync_copy(dst_hbm.at[pl.ds(r0, ch)], idst)

            cps = [pltpu.async_copy(as_sp.at[isrc.at[j]],
                                    asg.at[pl.ds(j * 128, 128)], sem)
                   for j in range(ch)]
            for cp in cps:
                cp.wait()
            cps = [pltpu.async_copy(ad_sp.at[idst.at[j]],
                                    adg.at[pl.ds(j * 128, 128)], sem)
                   for j in range(ch)]
            for cp in cps:
                cp.wait()

            def cw(m, _):
                e = asg[pl.ds(m * _LANES, _LANES)] + adg[pl.ds(m * _LANES, _LANES)]
                e = jnp.where(e < 0.0, e * 0.2, e)
                wbuf[pl.ds(m * _LANES, _LANES)] = jnp.exp(e)
                return 0
            lax.fori_loop(0, ch * 8, cw, 0)

            for j in range(ch):
                pltpu.sync_copy(wbuf.at[pl.ds(j * 128, 128)],
                                den.at[idst.at[j]], add=True)

            for j in range(ch):
                def ci(m, _, j=j):
                    irow[j, pl.ds(m * _LANES, _LANES)] = (
                        isrc[j, pl.ds(m * _LANES, _LANES)] * 2 + c)
                    return 0
                lax.fori_loop(0, 8, ci, 0)

            cps = [pltpu.async_copy(hrows_hbm.at[irow.at[j]],
                                    rows.at[pl.ds(j * 128, 128)], sem)
                   for j in range(ch)]
            for cp in cps:
                cp.wait()

            def sc_body(e2, _):
                rows[e2, :] = rows[e2, :] * wbuf[e2]
                return 0
            lax.fori_loop(0, ch * 128, sc_body, 0)

            for j in range(ch):
                pltpu.sync_copy(rows.at[pl.ds(j * 128, 128)],
                                acc.at[idst.at[j]], add=True)
            return 0
        lax.fori_loop(0, n_outer, outer, 0)
        plsc.subcore_barrier()

        off2 = jnp.minimum(s * tw, n - tw)
        pltpu.sync_copy(acc.at[pl.ds(off2, tw)],
                        num_out.at[pl.ds(c * n + off2, tw)])

        @pl.when(c == 0)
        def _():
            pltpu.sync_copy(den.at[pl.ds(off2, tw)], den_out.at[pl.ds(off2, tw)])

    return k(src2d, dst2d, as1, ad1, hrows)


def _sc2(src2d, dst2d, h2, asc, adc, n, n_pad, tw):
    rows_tot = src2d.shape[0]
    rpt = rows_tot // 32
    ch = 17
    n_outer = rpt // ch
    assert n_outer * ch == rpt
    mesh = plsc.VectorSubcoreMesh(core_axis_name="c", subcore_axis_name="s")

    @functools.partial(
        pl.kernel, mesh=mesh,
        out_type=[jax.ShapeDtypeStruct((2 * n,), _F32),
                  jax.ShapeDtypeStruct((2 * n,), _F32)],
        scratch_types=[
            pltpu.VMEM_SHARED((n_pad,), _F32),
            pltpu.VMEM_SHARED((n_pad,), _F32),
            pltpu.VMEM_SHARED((n_pad,), _F32),
            pltpu.VMEM((ch, 128), _I32),
            pltpu.VMEM((ch, 128), _I32),
            pltpu.VMEM((ch * 128,), _F32),
            pltpu.VMEM((ch * 128,), _F32),
            pltpu.VMEM((ch * 128,), _F32),
            pltpu.VMEM((ch * 128,), _F32),
            pltpu.VMEM((_LANES,), _F32),
            pltpu.VMEM((_LANES,), _F32),
            pltpu.VMEM((tw,), _F32),
            pltpu.SemaphoreType.DMA,
        ])
    def k(src_hbm, dst_hbm, h2_hbm, asc_hbm, adc_hbm, num_out, den_out,
          h2_sp, num_sp, den_sp, isrc, idst, gsrc, gdst, wbuf, wgbuf,
          ascv, adcv, z1, sem):
        c = lax.axis_index("c")
        s = lax.axis_index("s")
        wid = s * 2 + c
        zv = jnp.zeros((_LANES,), _F32)

        def z1_body(i, _):
            z1[pl.ds(i * _LANES, _LANES)] = zv
            return 0
        lax.fori_loop(0, tw // _LANES, z1_body, 0)

        base = s * tw
        pltpu.sync_copy(z1, num_sp.at[pl.ds(base, tw)])
        pltpu.sync_copy(z1, den_sp.at[pl.ds(base, tw)])

        off = jnp.minimum(s * tw, n - tw)
        pltpu.sync_copy(h2_hbm.at[pl.ds(off, tw)], h2_sp.at[pl.ds(off, tw)])

        @pl.when(s == 0)
        def _():
            pad = n_pad - n
            pltpu.sync_copy(z1.at[pl.ds(0, pad)], h2_sp.at[pl.ds(n, pad)])

        pltpu.sync_copy(asc_hbm, ascv)
        pltpu.sync_copy(adc_hbm, adcv)
        plsc.subcore_barrier()

        av = ascv[...]
        dv = adcv[...]

        def outer(o, _):
            r0 = wid * rpt + o * ch
            pltpu.sync_copy(src_hbm.at[pl.ds(r0, ch)], isrc)
            pltpu.sync_copy(dst_hbm.at[pl.ds(r0, ch)], idst)

            cps = [pltpu.async_copy(h2_sp.at[isrc.at[j]],
                                    gsrc.at[pl.ds(j * 128, 128)], sem)
                   for j in range(ch)]
            for cp in cps:
                cp.wait()
            cps = [pltpu.async_copy(h2_sp.at[idst.at[j]],
                                    gdst.at[pl.ds(j * 128, 128)], sem)
                   for j in range(ch)]
            for cp in cps:
                cp.wait()

            def cw(m, _):
                g = gsrc[pl.ds(m * _LANES, _LANES)]
                e = g * av + gdst[pl.ds(m * _LANES, _LANES)] * dv
                e = jnp.where(e < 0.0, e * 0.2, e)
                w = jnp.exp(e)
                wbuf[pl.ds(m * _LANES, _LANES)] = w
                wgbuf[pl.ds(m * _LANES, _LANES)] = w * g
                return 0
            lax.fori_loop(0, ch * 8, cw, 0)

            for j in range(ch):
                pltpu.sync_copy(wbuf.at[pl.ds(j * 128, 128)],
                                den_sp.at[idst.at[j]], add=True)
                pltpu.sync_copy(wgbuf.at[pl.ds(j * 128, 128)],
                                num_sp.at[idst.at[j]], add=True)
            return 0
        lax.fori_loop(0, n_outer, outer, 0)
        plsc.subcore_barrier()

        off2 = jnp.minimum(s * tw, n - tw)
        pltpu.sync_copy(num_sp.at[pl.ds(off2, tw)],
                        num_out.at[pl.ds(c * n + off2, tw)])
        pltpu.sync_copy(den_sp.at[pl.ds(off2, tw)],
                        den_out.at[pl.ds(c * n + off2, tw)])

    return k(src2d, dst2d, h2, asc, adc)


def kernel(x, edge_index, W1, a_src1, a_dst1, b1, W2, a_src2, a_dst2, b2):
    n = x.shape[0]
    e = edge_index.shape[1]

    rows = -(-e // 128)
    rows_pad = -(-rows // 32) * 32
    e_pad = rows_pad * 128
    npad_e = e_pad - e

    n_pad = -(-n // (16 * 8)) * (16 * 8)
    tw = n_pad // 16
    pad_n = n_pad - n

    src = edge_index[0]
    dst = edge_index[1]
    if npad_e:
        fill = jnp.arange(npad_e, dtype=_I32)
        src = jnp.concatenate([src, fill % 64])
        dst = jnp.concatenate([dst, n + (fill % min(pad_n, 96))])
    src2d = src.reshape(rows_pad, 128)
    dst2d = dst.reshape(rows_pad, 128)

    h1, as1, ad1 = _tc_a(x, W1, a_src1, a_dst1)
    hrows = h1.reshape(2 * n, 16)

    num1, den1 = _sc1(src2d, dst2d, as1, ad1, hrows, n, n_pad, tw)
    h2 = _tc_b(num1.reshape(2, n, 16), den1, b1, W2)

    asc = jnp.broadcast_to(a_src2, (_LANES,))
    adc = jnp.broadcast_to(a_dst2, (_LANES,))
    num2, den2 = _sc2(src2d, dst2d, h2, asc, adc, n, n_pad, tw)

    return _tc_c(num2.reshape(2, n), den2.reshape(2, n), b2)


# speedup vs baseline: 70.9440x; 70.9440x over previous
import functools

import jax
import jax.numpy as jnp
from jax import lax
from jax.experimental import pallas as pl
from jax.experimental.pallas import tpu as pltpu
from jax.experimental.pallas import tpu_sc as plsc

_F32 = jnp.float32
_I32 = jnp.int32


def _tc_a(x, W1, a1, a2):
    n, f_in = x.shape
    hid = W1.shape[1]
    blk = 5000
    grid = n // blk

    def body(x_ref, w_ref, s_ref, d_ref, h_ref, as_ref, ad_ref):
        h = lax.dot_general(x_ref[...], w_ref[...], (((1,), (0,)), ((), ())),
                            preferred_element_type=_F32)
        h_ref[...] = h
        as_ref[...] = jnp.sum(h * s_ref[...][None, :], axis=1)[:, None]
        ad_ref[...] = jnp.sum(h * d_ref[...][None, :], axis=1)[:, None]

    return pl.pallas_call(
        body,
        grid=(grid,),
        in_specs=[pl.BlockSpec((blk, f_in), lambda i: (i, 0)),
                  pl.BlockSpec((f_in, hid), lambda i: (0, 0)),
                  pl.BlockSpec((hid,), lambda i: (0,)),
                  pl.BlockSpec((hid,), lambda i: (0,))],
        out_specs=[pl.BlockSpec((blk, hid), lambda i: (i, 0)),
                   pl.BlockSpec((blk, 1), lambda i: (i, 0)),
                   pl.BlockSpec((blk, 1), lambda i: (i, 0))],
        out_shape=[jax.ShapeDtypeStruct((n, hid), _F32),
                   jax.ShapeDtypeStruct((n, 1), _F32),
                   jax.ShapeDtypeStruct((n, 1), _F32)],
    )(x, W1, a1, a2)


def _tc_b(num, dena, denb, b1, W2):
    n = dena.shape[0]
    hid = b1.shape[0]
    blk = 5000
    grid = n // blk

    def body(num_ref, da_ref, db_ref, b_ref, w_ref, h2_ref):
        hcat = jnp.concatenate([num_ref[0], num_ref[1]], axis=1)
        d = da_ref[...] + db_ref[...] + 1e-16
        hr = jnp.maximum(hcat / d + b_ref[...][None, :], 0.0)
        h2_ref[...] = jnp.sum(hr * w_ref[...][:, 0][None, :], axis=1)[:, None]

    dspec = pl.BlockSpec((blk, 1), lambda i: (i, 0))
    return pl.pallas_call(
        body,
        grid=(grid,),
        in_specs=[pl.BlockSpec((2, blk, 16), lambda i: (0, i, 0)),
                  dspec, dspec,
                  pl.BlockSpec((hid,), lambda i: (0,)),
                  pl.BlockSpec((hid, 1), lambda i: (0, 0))],
        out_specs=[pl.BlockSpec((blk, 1), lambda i: (i, 0))],
        out_shape=[jax.ShapeDtypeStruct((n, 1), _F32)],
    )(num, dena.reshape(n, 1), denb.reshape(n, 1), b1, W2)[0].reshape(n)


def _tc_c(na, nb, da, db, b2):
    n = na.shape[0]
    blk = 5000
    grid = n // blk

    def body(na_ref, nb_ref, da_ref, db_ref, b_ref, o_ref):
        v = (na_ref[...] + nb_ref[...]) / (da_ref[...] + db_ref[...] + 1e-16)
        o_ref[...] = v + b_ref[...][None, :]

    spec = pl.BlockSpec((blk, 1), lambda i: (i, 0))
    return pl.pallas_call(
        body,
        grid=(grid,),
        in_specs=[spec, spec, spec, spec,
                  pl.BlockSpec((1,), lambda i: (0,))],
        out_specs=[spec],
        out_shape=[jax.ShapeDtypeStruct((n, 1), _F32)],
    )(na, nb, da, db, b2)[0]


_LANES = 16


def _sc1a(src2d, dst2d, as1, ad1, n, n_pad, tw, e_pad):
    rows_tot = src2d.shape[0]
    rpt = rows_tot // 32
    ch = 8
    n_outer = rpt // ch
    assert n_outer * ch == rpt and rpt % 8 == 0
    mesh = plsc.VectorSubcoreMesh(core_axis_name="c", subcore_axis_name="s")

    @functools.partial(
        pl.kernel, mesh=mesh,
        compiler_params=pltpu.CompilerParams(use_tc_tiling_on_sc=False),
        out_type=[jax.ShapeDtypeStruct((e_pad,), _F32),
                  jax.ShapeDtypeStruct((2 * n,), _F32)],
        scratch_types=[
            pltpu.VMEM_SHARED((n_pad,), _F32),
            pltpu.VMEM_SHARED((n_pad,), _F32),
            pltpu.VMEM_SHARED((n_pad,), _F32),
            pltpu.VMEM((ch, 128), _I32),
            pltpu.VMEM((ch, 128), _I32),
            pltpu.VMEM((ch * 128,), _F32),
            pltpu.VMEM((ch * 128,), _F32),
            pltpu.VMEM((ch * 128,), _F32),
            pltpu.VMEM((tw,), _F32),
            pltpu.VMEM((tw,), _F32),
            pltpu.SemaphoreType.DMA,
        ])
    def k(src_hbm, dst_hbm, as_hbm, ad_hbm, w_out, den_out,
          den, as_sp, ad_sp, isrc, idst, asg, adg, wbuf, z1, stg, sem):
        c = lax.axis_index("c")
        s = lax.axis_index("s")
        wid = s * 2 + c
        zv = jnp.zeros((_LANES,), _F32)

        def z1_body(i, _):
            z1[pl.ds(i * _LANES, _LANES)] = zv
            return 0
        lax.fori_loop(0, tw // _LANES, z1_body, 0)

        pltpu.sync_copy(z1, den.at[pl.ds(s * tw, tw)])

        off = jnp.minimum(s * tw, n - tw)
        pltpu.sync_copy(as_hbm.at[pl.ds(off, tw)], stg)
        pltpu.sync_copy(stg, as_sp.at[pl.ds(off, tw)])
        pltpu.sync_copy(ad_hbm.at[pl.ds(off, tw)], stg)
        pltpu.sync_copy(stg, ad_sp.at[pl.ds(off, tw)])

        @pl.when(s == 0)
        def _():
            pad = n_pad - n
            pltpu.sync_copy(z1.at[pl.ds(0, pad)], as_sp.at[pl.ds(n, pad)])
            pltpu.sync_copy(z1.at[pl.ds(0, pad)], ad_sp.at[pl.ds(n, pad)])

        plsc.subcore_barrier()

        def outer(o, _):
            r0 = wid * rpt + o * ch
            pltpu.sync_copy(src_hbm.at[pl.ds(r0, ch)], isrc)
            pltpu.sync_copy(dst_hbm.at[pl.ds(r0, ch)], idst)

            cps = [pltpu.async_copy(as_sp.at[isrc.at[j]],
                                    asg.at[pl.ds(j * 128, 128)], sem)
                   for j in range(ch)]
            for cp in cps:
                cp.wait()
            cps = [pltpu.async_copy(ad_sp.at[idst.at[j]],
                                    adg.at[pl.ds(j * 128, 128)], sem)
                   for j in range(ch)]
            for cp in cps:
                cp.wait()

            def cw(m, _):
                e = asg[pl.ds(m * _LANES, _LANES)] + adg[pl.ds(m * _LANES, _LANES)]
                e = jnp.where(e < 0.0, e * 0.2, e)
                wbuf[pl.ds(m * _LANES, _LANES)] = jnp.exp(e)
                return 0
            lax.fori_loop(0, ch * 8, cw, 0)

            pltpu.sync_copy(wbuf, w_out.at[pl.ds(r0 * 128, ch * 128)])
            for j in range(ch):
                pltpu.sync_copy(wbuf.at[pl.ds(j * 128, 128)],
                                den.at[idst.at[j]], add=True)
            return 0
        lax.fori_loop(0, n_outer, outer, 0)
        plsc.subcore_barrier()

        off2 = jnp.minimum(s * tw, n - tw)
        pltpu.sync_copy(den.at[pl.ds(off2, tw)], stg)
        pltpu.sync_copy(stg, den_out.at[pl.ds(c * n + off2, tw)])

    return k(src2d, dst2d, as1, ad1)


def _sc1b(src2d, dst2d, wlin, hrows, n, n_pad, tw):
    rows_tot = src2d.shape[0]
    rpt = rows_tot // 16
    ch = 8
    n_outer = rpt // ch
    assert n_outer * ch == rpt and rpt % 8 == 0
    zr = tw // 17
    mesh = plsc.VectorSubcoreMesh(core_axis_name="c", subcore_axis_name="s")

    @functools.partial(
        pl.kernel, mesh=mesh,
        compiler_params=pltpu.CompilerParams(use_tc_tiling_on_sc=False),
        out_type=[jax.ShapeDtypeStruct((2 * n, 16), _F32)],
        scratch_types=[
            pltpu.VMEM_SHARED((n_pad, 16), _F32),
            pltpu.VMEM((ch, 128), _I32),
            pltpu.VMEM((ch, 128), _I32),
            pltpu.VMEM((ch, 128), _I32),
            pltpu.VMEM((ch * 128,), _F32),
            pltpu.VMEM((ch * 128, 16), _F32),
            pltpu.VMEM((zr, 16), _F32),
            pltpu.SemaphoreType.DMA,
        ])
    def k(src_hbm, dst_hbm, w_hbm, hrows_hbm, num_out,
          acc, isrc, idst, irow, wbuf, rows, z2, sem):
        c = lax.axis_index("c")
        s = lax.axis_index("s")
        zv = jnp.zeros((_LANES,), _F32)

        def z2_body(i, _):
            z2[i, :] = zv
            return 0
        lax.fori_loop(0, zr, z2_body, 0)

        base = s * tw
        for b in range(17):
            pltpu.sync_copy(z2, acc.at[pl.ds(base + b * zr, zr)])
        plsc.subcore_barrier()

        def outer(o, _):
            r0 = s * rpt + o * ch
            pltpu.sync_copy(src_hbm.at[pl.ds(r0, ch)], isrc)
            pltpu.sync_copy(dst_hbm.at[pl.ds(r0, ch)], idst)
            pltpu.sync_copy(w_hbm.at[pl.ds(r0 * 128, ch * 128)], wbuf)

            for j in range(ch):
                def ci(m, _, j=j):
                    irow[j, pl.ds(m * _LANES, _LANES)] = (
                        isrc[j, pl.ds(m * _LANES, _LANES)] * 2 + c)
                    return 0
                lax.fori_loop(0, 8, ci, 0)

            cps = [pltpu.async_copy(hrows_hbm.at[irow.at[j]],
                                    rows.at[pl.ds(j * 128, 128)], sem)
                   for j in range(ch)]
            for cp in cps:
                cp.wait()

            def sc_body(m, _):
                rb = m * _LANES
                w16 = wbuf[pl.ds(rb, _LANES)]
                for i in range(_LANES):
                    wi = lax.slice(w16, (i,), (i + 1,))
                    wv = lax.broadcast_in_dim(wi, (_LANES,), (0,))
                    rows[rb + i, :] = rows[rb + i, :] * wv
                return 0
            lax.fori_loop(0, ch * 8, sc_body, 0)

            for j in range(ch):
                pltpu.sync_copy(rows.at[pl.ds(j * 128, 128)],
                                acc.at[idst.at[j]], add=True)
            return 0
        lax.fori_loop(0, n_outer, outer, 0)
        plsc.subcore_barrier()

        off2 = jnp.minimum(s * tw, n - tw)
        for b in range(17):
            pltpu.sync_copy(acc.at[pl.ds(off2 + b * zr, zr)], z2)
            pltpu.sync_copy(z2,
                            num_out.at[pl.ds(c * n + off2 + b * zr, zr)])

    return k(src2d, dst2d, wlin, hrows)


def _sc2(src2d, dst2d, h2, asc, adc, n, n_pad, tw):
    rows_tot = src2d.shape[0]
    rpt = rows_tot // 32
    ch = 8
    n_outer = rpt // ch
    assert n_outer * ch == rpt and rpt % 8 == 0
    mesh = plsc.VectorSubcoreMesh(core_axis_name="c", subcore_axis_name="s")

    @functools.partial(
        pl.kernel, mesh=mesh,
        compiler_params=pltpu.CompilerParams(use_tc_tiling_on_sc=False),
        out_type=[jax.ShapeDtypeStruct((2 * n,), _F32),
                  jax.ShapeDtypeStruct((2 * n,), _F32)],
        scratch_types=[
            pltpu.VMEM_SHARED((n_pad,), _F32),
            pltpu.VMEM_SHARED((n_pad,), _F32),
            pltpu.VMEM_SHARED((n_pad,), _F32),
            pltpu.VMEM((ch, 128), _I32),
            pltpu.VMEM((ch, 128), _I32),
            pltpu.VMEM((ch * 128,), _F32),
            pltpu.VMEM((ch * 128,), _F32),
            pltpu.VMEM((ch * 128,), _F32),
            pltpu.VMEM((ch * 128,), _F32),
            pltpu.VMEM((_LANES,), _F32),
            pltpu.VMEM((_LANES,), _F32),
            pltpu.VMEM((tw,), _F32),
            pltpu.VMEM((tw,), _F32),
            pltpu.SemaphoreType.DMA,
        ])
    def k(src_hbm, dst_hbm, h2_hbm, asc_hbm, adc_hbm, num_out, den_out,
          h2_sp, num_sp, den_sp, isrc, idst, gsrc, gdst, wbuf, wgbuf,
          ascv, adcv, z1, stg, sem):
        c = lax.axis_index("c")
        s = lax.axis_index("s")
        wid = s * 2 + c
        zv = jnp.zeros((_LANES,), _F32)

        def z1_body(i, _):
            z1[pl.ds(i * _LANES, _LANES)] = zv
            return 0
        lax.fori_loop(0, tw // _LANES, z1_body, 0)

        base = s * tw
        pltpu.sync_copy(z1, num_sp.at[pl.ds(base, tw)])
        pltpu.sync_copy(z1, den_sp.at[pl.ds(base, tw)])

        off = jnp.minimum(s * tw, n - tw)
        pltpu.sync_copy(h2_hbm.at[pl.ds(off, tw)], stg)
        pltpu.sync_copy(stg, h2_sp.at[pl.ds(off, tw)])

        @pl.when(s == 0)
        def _():
            pad = n_pad - n
            pltpu.sync_copy(z1.at[pl.ds(0, pad)], h2_sp.at[pl.ds(n, pad)])

        pltpu.sync_copy(asc_hbm, ascv)
        pltpu.sync_copy(adc_hbm, adcv)
        plsc.subcore_barrier()

        av = ascv[...]
        dv = adcv[...]

        def outer(o, _):
            r0 = wid * rpt + o * ch
            pltpu.sync_copy(src_hbm.at[pl.ds(r0, ch)], isrc)
            pltpu.sync_copy(dst_hbm.at[pl.ds(r0, ch)], idst)

            cps = [pltpu.async_copy(h2_sp.at[isrc.at[j]],
                                    gsrc.at[pl.ds(j * 128, 128)], sem)
                   for j in range(ch)]
            for cp in cps:
                cp.wait()
            cps = [pltpu.async_copy(h2_sp.at[idst.at[j]],
                                    gdst.at[pl.ds(j * 128, 128)], sem)
                   for j in range(ch)]
            for cp in cps:
                cp.wait()

            def cw(m, _):
                g = gsrc[pl.ds(m * _LANES, _LANES)]
                e = g * av + gdst[pl.ds(m * _LANES, _LANES)] * dv
                e = jnp.where(e < 0.0, e * 0.2, e)
                w = jnp.exp(e)
                wbuf[pl.ds(m * _LANES, _LANES)] = w
                wgbuf[pl.ds(m * _LANES, _LANES)] = w * g
                return 0
            lax.fori_loop(0, ch * 8, cw, 0)

            for j in range(ch):
                pltpu.sync_copy(wbuf.at[pl.ds(j * 128, 128)],
                                den_sp.at[idst.at[j]], add=True)
                pltpu.sync_copy(wgbuf.at[pl.ds(j * 128, 128)],
                                num_sp.at[idst.at[j]], add=True)
            return 0
        lax.fori_loop(0, n_outer, outer, 0)
        plsc.subcore_barrier()

        off2 = jnp.minimum(s * tw, n - tw)
        pltpu.sync_copy(num_sp.at[pl.ds(off2, tw)], stg)
        pltpu.sync_copy(stg, num_out.at[pl.ds(c * n + off2, tw)])
        pltpu.sync_copy(den_sp.at[pl.ds(off2, tw)], stg)
        pltpu.sync_copy(stg, den_out.at[pl.ds(c * n + off2, tw)])

    return k(src2d, dst2d, h2, asc, adc)


def kernel(x, edge_index, W1, a_src1, a_dst1, b1, W2, a_src2, a_dst2, b2):
    n = x.shape[0]
    e = edge_index.shape[1]

    rows = -(-e // 128)
    rows_pad = -(-rows // 128) * 128
    e_pad = rows_pad * 128
    npad_e = e_pad - e

    n_pad = -(-n // (16 * 8)) * (16 * 8)
    tw = n_pad // 16
    pad_n = n_pad - n

    src = edge_index[0]
    dst = edge_index[1]
    if npad_e:
        fill = jnp.arange(npad_e, dtype=_I32)
        src = jnp.concatenate([src, fill % 64])
        dst = jnp.concatenate([dst, n + (fill % min(pad_n, 96))])
    src2d = src.reshape(rows_pad, 128)
    dst2d = dst.reshape(rows_pad, 128)

    h1, as1, ad1 = _tc_a(x, W1, a_src1, a_dst1)
    hrows = h1.reshape(2 * n, 16)

    wlin, den1 = _sc1a(src2d, dst2d, as1.reshape(n), ad1.reshape(n),
                       n, n_pad, tw, e_pad)
    num1 = _sc1b(src2d, dst2d, wlin, hrows, n, n_pad, tw)[0]
    h2 = _tc_b(num1.reshape(2, n, 16), den1[:n], den1[n:], b1, W2)

    asc = jnp.broadcast_to(a_src2, (_LANES,))
    adc = jnp.broadcast_to(a_dst2, (_LANES,))
    num2, den2 = _sc2(src2d, dst2d, h2, asc, adc, n, n_pad, tw)

    return _tc_c(num2[:n].reshape(n, 1), num2[n:].reshape(n, 1),
                 den2[:n].reshape(n, 1), den2[n:].reshape(n, 1), b2)
